# baseline (speedup 1.0000x reference)
import functools

import jax
import jax.numpy as jnp
from jax import lax
from jax.experimental import pallas as pl
from jax.experimental.pallas import tpu as pltpu
from jax.experimental.pallas import tpu_sc as plsc

_PRIME = 31
_NMIN, _NMAX = 3, 20
_NT = _NMAX - _NMIN + 1
_TBL = 16384
_D = 64
_B, _L = 4, 8192
_NPOS = _B * _L
_INV = 1.0 / (_NT + 1)

_NW = 32
_W = 1024
_P = 512
_G = 128


def _hash_body(byte_ref, ids_ref):
    b = byte_ref[...]
    is_dna = (b >= 1) & (b <= 4)
    safe = jnp.where(is_dna, b - 1, 0)
    invalid = jnp.where(is_dna, 0, 1)
    first = jnp.where(
        lax.broadcasted_iota(jnp.int32, b.shape, 1) == 0, 1, 0)
    h = safe
    bad = invalid
    for n in range(2, _NMAX + 1):
        h = (pltpu.roll(h, 1, 1) * _PRIME + safe) & (_TBL - 1)
        bad = pltpu.roll(bad, 1, 1) | first | invalid
        if n >= _NMIN:
            ids_ref[n - _NMIN, :, :] = jnp.where(bad == 1, _TBL, h)


_hash_call = pl.pallas_call(
    _hash_body,
    out_shape=jax.ShapeDtypeStruct((_NT, _B, _L), jnp.int32),
)


def _sc_body(tab_ref, ids_ref, out_ref,
             ids_v, out_v, fixpos, rows0, rows1, mrows, suffix, dslot,
             sem_s, sem_m, sem_g0, sem_g1, sem_o0, sem_o1):
    i16 = lax.iota(jnp.int32, 16)
    wid = lax.axis_index("s") * 2 + lax.axis_index("c")
    wbase = wid * _W
    bb = wbase // _L
    l0 = wbase % _L

    slab_cp = pltpu.async_copy(ids_ref.at[:, bb, pl.ds(l0, _W)], ids_v,
                               sem_s)
    mcps = [pltpu.async_copy(tab_ref.at[i, pl.ds(_TBL, 1), :],
                             mrows.at[pl.ds(i, 1), :], sem_m)
            for i in range(_NT)]
    for cp in mcps:
        cp.wait()

    zf = jnp.zeros((16,), jnp.float32)
    for k in range(4):
        suffix[pl.ds(_NT * _D + 16 * k, 16)] = zf
    mhi = jnp.full((16,), -65536, jnp.int32)
    for i in range(_NT - 1, -1, -1):
        for k2 in range(2):
            w = mrows[i, pl.ds(16 * k2, 16)]
            ev = plsc.bitcast(lax.shift_left(w, 16), jnp.float32)
            od = plsc.bitcast(w & mhi, jnp.float32)
            idx_e = 32 * k2 + 2 * i16
            pe = plsc.load_gather(suffix, [(i + 1) * _D + idx_e])
            plsc.store_scatter(suffix, [i * _D + idx_e], pe + ev)
            po = plsc.load_gather(suffix, [(i + 1) * _D + idx_e + 1])
            plsc.store_scatter(suffix, [i * _D + idx_e + 1], po + od)
    cst = [suffix[pl.ds(16 * k, 16)] * _INV for k in range(4)]

    def _lane(ref, f):
        off = f & (-16)
        lane = f & 15
        v = ref[pl.ds(off, 16)]
        return jnp.sum(jnp.where(i16 == lane, v, 0))

    def _cand(jl):
        fullj = jnp.broadcast_to(jl, (16,))
        v1 = plsc.load_gather(ids_v, [i16, fullj])
        m1 = v1 != _TBL
        v2 = plsc.load_gather(ids_v, [jnp.minimum(i16 + 16, _NT - 1), fullj])
        m2 = (v2 != _TBL) & (i16 < (_NT - 16))
        return v1, m1, v2, m2

    slab_cp.wait()

    def scan_body(m, nfix):
        v = ids_v[0, pl.ds(16 * m, 16)]
        msk = v != _TBL
        plsc.store_compressed(fixpos.at[pl.ds(nfix, 16)],
                              i16 + 16 * m, mask=msk)
        return nfix + jnp.sum(jnp.where(msk, 1, 0))
    n0 = lax.fori_loop(0, _P // 16, scan_body, 0)
    ntot = lax.fori_loop(_P // 16, _W // 16, scan_body, n0)

    def _fire(fstart, limit, rows_v, gsem):
        def body(st):
            f, goff = st
            jl = _lane(fixpos, f)
            v1, m1, v2, m2 = _cand(jl)
            c1 = jnp.sum(jnp.where(m1, 1, 0))
            c2 = jnp.sum(jnp.where(m2, 1, 0))

            def fire1(r, carry):
                idr = jnp.sum(jnp.where(i16 == r, v1, 0))
                pltpu.async_copy(tab_ref.at[r, pl.ds(idr, 1), :],
                                 rows_v.at[pl.ds(goff + r, 1), :], gsem)
                return carry
            lax.fori_loop(0, c1, fire1, 0)

            def fire2(r, carry):
                idr = jnp.sum(jnp.where(i16 == r, v2, 0))
                pltpu.async_copy(tab_ref.at[16 + r, pl.ds(idr, 1), :],
                                 rows_v.at[pl.ds(goff + c1 + r, 1), :], gsem)
                return carry
            lax.fori_loop(0, c2, fire2, 0)
            return f + 1, goff + c1 + c2

        def cond(st):
            f, goff = st
            return (f < limit) & (goff <= _G - _NT)
        return lax.while_loop(cond, body, (fstart, 0))

    def _drain(count, gsem):
        def body(r, carry):
            pltpu.make_async_copy(tab_ref.at[0, pl.ds(0, 1), :], dslot,
                                  gsem).wait()
            return carry
        lax.fori_loop(0, count, body, 0)

    f1_0, k0 = _fire(0, n0, rows0, sem_g0)
    f1_1, k1 = _fire(n0, ntot, rows1, sem_g1)

    def _p2(fstart, fend, roff0, rows_v, choff):
        def body(st):
            f, roff = st
            jl = _lane(fixpos, f)
            v1, m1, v2, m2 = _cand(jl)
            c = jnp.sum(jnp.where(m1, 1, 0)) + jnp.sum(jnp.where(m2, 1, 0))
            mhi2 = jnp.full((16,), -65536, jnp.int32)
            accs = tuple(
                plsc.load_gather(suffix,
                                 [c * _D + 32 * k2 + 2 * i16 + par])
                for k2 in range(2) for par in range(2))

            def inner(r, accs):
                a0, a1, a2, a3 = accs
                w0 = rows_v[roff + r, pl.ds(0, 16)]
                w1 = rows_v[roff + r, pl.ds(16, 16)]
                a0 = a0 + plsc.bitcast(lax.shift_left(w0, 16), jnp.float32)
                a1 = a1 + plsc.bitcast(w0 & mhi2, jnp.float32)
                a2 = a2 + plsc.bitcast(lax.shift_left(w1, 16), jnp.float32)
                a3 = a3 + plsc.bitcast(w1 & mhi2, jnp.float32)
                return a0, a1, a2, a3
            accs = lax.fori_loop(0, c, inner, accs)
            fj = jnp.broadcast_to(jl - choff, (16,))
            for kk, (k2, par) in enumerate(
                    [(0, 0), (0, 1), (1, 0), (1, 1)]):
                plsc.store_scatter(out_v, [fj, 32 * k2 + 2 * i16 + par],
                                   accs[kk] * _INV)
            return f + 1, roff + c
        return lax.while_loop(lambda st: st[0] < fend, body,
                              (fstart, roff0))

    chunk_meta = [
        (0, f1_0, n0, k0, rows0, sem_g0, sem_o0),
        (n0, f1_1, ntot, k1, rows1, sem_g1, sem_o1),
    ]
    out_copies = []
    for ch in range(2):
        s_ch, f1_ch, e_ch, k_ch, rv, gsem, osem = chunk_meta[ch]
        if ch == 1:
            out_copies[0].wait()

        def fill_body(q, carry):
            for u in range(4):
                for k in range(4):
                    out_v[q * 4 + u, pl.ds(16 * k, 16)] = cst[k]
            return carry
        lax.fori_loop(0, _P // 4, fill_body, 0)

        _drain(k_ch, gsem)
        fdone, _ = _p2(s_ch, f1_ch, 0, rv, ch * _P)

        def rem_body(f):
            fn, kk = _fire(f, e_ch, rv, gsem)
            _drain(kk, gsem)
            fn2, _ = _p2(f, fn, 0, rv, ch * _P)
            return fn2
        lax.while_loop(lambda f: f < e_ch, rem_body, fdone)

        out_copies.append(
            pltpu.async_copy(out_v,
                             out_ref.at[bb, pl.ds(l0 + ch * _P, _P), :],
                             osem))
    out_copies[1].wait()


@functools.cache
def _sc_call():
    return pl.kernel(
        _sc_body,
        out_type=jax.ShapeDtypeStruct((_B, _L, _D), jnp.float32),
        mesh=plsc.VectorSubcoreMesh(core_axis_name="c", subcore_axis_name="s"),
        compiler_params=pltpu.CompilerParams(needs_layout_passes=False),
        scratch_types=[
            pltpu.VMEM((_NT, _W), jnp.int32),
            pltpu.VMEM((_P, _D), jnp.float32),
            pltpu.VMEM((_W + 16,), jnp.int32),
            pltpu.VMEM((_G, _D // 2), jnp.int32),
            pltpu.VMEM((_G, _D // 2), jnp.int32),
            pltpu.VMEM((_NT, _D // 2), jnp.int32),
            pltpu.VMEM(((_NT + 1) * _D,), jnp.float32),
            pltpu.VMEM((1, _D // 2), jnp.int32),
            pltpu.SemaphoreType.DMA,
            pltpu.SemaphoreType.DMA,
            pltpu.SemaphoreType.DMA,
            pltpu.SemaphoreType.DMA,
            pltpu.SemaphoreType.DMA,
            pltpu.SemaphoreType.DMA,
        ],
    )


def kernel(byte_ids, tables):
    byte_ids = byte_ids.astype(jnp.int32)
    ids = _hash_call(byte_ids)
    tab16 = tables.astype(jnp.bfloat16)
    tabi = jax.lax.bitcast_convert_type(
        tab16.reshape(_NT, _TBL + 1, _D // 2, 2), jnp.int32)
    return _sc_call()(tabi, ids).astype(tables.dtype)


# speedup vs baseline: 3.4022x; 3.4022x over previous
import functools

import jax
import jax.numpy as jnp
from jax import lax
from jax.experimental import pallas as pl
from jax.experimental.pallas import tpu as pltpu
from jax.experimental.pallas import tpu_sc as plsc

_PRIME = 31
_NMIN, _NMAX = 3, 20
_NT = _NMAX - _NMIN + 1
_TBL = 16384
_D = 64
_B, _L = 4, 8192
_NPOS = _B * _L
_INV = 1.0 / (_NT + 1)

_NW = 32
_W = 1024
_P = 512
_G = 128


def _hash_body(byte_ref, ids_ref):
    b = byte_ref[...]
    is_dna = (b >= 1) & (b <= 4)
    safe = jnp.where(is_dna, b - 1, 0)
    invalid = jnp.where(is_dna, 0, 1)
    first = jnp.where(
        lax.broadcasted_iota(jnp.int32, b.shape, 1) == 0, 1, 0)
    h = safe
    bad = invalid
    for n in range(2, _NMAX + 1):
        h = (pltpu.roll(h, 1, 1) * _PRIME + safe) & (_TBL - 1)
        bad = pltpu.roll(bad, 1, 1) | first | invalid
        if n >= _NMIN:
            ids_ref[n - _NMIN, :, :] = jnp.where(bad == 1, _TBL, h)


_hash_call = pl.pallas_call(
    _hash_body,
    out_shape=jax.ShapeDtypeStruct((_NT, _B, _L), jnp.int32),
)


def _sc_body(tab_ref, ids_ref, out_ref,
             ids_v, out_v, fixpos, rows0, rows1, mrows, suffix, dslot,
             sem_s, sem_m, sem_g0, sem_g1, sem_o0, sem_o1):
    i16 = lax.iota(jnp.int32, 16)
    wid = lax.axis_index("s") * 2 + lax.axis_index("c")
    wbase = wid * _W
    bb = wbase // _L
    l0 = wbase % _L

    slab_cp = pltpu.async_copy(ids_ref.at[:, bb, pl.ds(l0, _W)], ids_v,
                               sem_s)
    mcps = [pltpu.async_copy(tab_ref.at[i, pl.ds(_TBL, 1), :],
                             mrows.at[pl.ds(i, 1), :], sem_m)
            for i in range(_NT)]
    for cp in mcps:
        cp.wait()

    zf = jnp.zeros((16,), jnp.float32)
    for k in range(4):
        suffix[pl.ds(_NT * _D + 16 * k, 16)] = zf
    for i in range(_NT - 1, -1, -1):
        for k in range(4):
            mr = mrows[i, pl.ds(16 * k, 16)]
            suffix[pl.ds(i * _D + 16 * k, 16)] = (
                suffix[pl.ds((i + 1) * _D + 16 * k, 16)] + mr)
    cst = [suffix[pl.ds(16 * k, 16)] * _INV for k in range(4)]

    def _lane(ref, f):
        off = f & (-16)
        lane = f & 15
        v = ref[pl.ds(off, 16)]
        return jnp.sum(jnp.where(i16 == lane, v, 0))

    def _cand(jl):
        fullj = jnp.broadcast_to(jl, (16,))
        v1 = plsc.load_gather(ids_v, [i16, fullj])
        m1 = v1 != _TBL
        v2 = plsc.load_gather(ids_v, [jnp.minimum(i16 + 16, _NT - 1), fullj])
        m2 = (v2 != _TBL) & (i16 < (_NT - 16))
        return v1, m1, v2, m2

    slab_cp.wait()

    def scan_body(m, nfix):
        v = ids_v[0, pl.ds(16 * m, 16)]
        msk = v != _TBL
        plsc.store_compressed(fixpos.at[pl.ds(nfix, 16)],
                              i16 + 16 * m, mask=msk)
        return nfix + jnp.sum(jnp.where(msk, 1, 0))
    n0 = lax.fori_loop(0, _P // 16, scan_body, 0)
    ntot = lax.fori_loop(_P // 16, _W // 16, scan_body, n0)

    def _fire(fstart, limit, rows_v, gsem):
        def body(st):
            f, goff = st
            jl = _lane(fixpos, f)
            v1, m1, v2, m2 = _cand(jl)
            c1 = jnp.sum(jnp.where(m1, 1, 0))
            c2 = jnp.sum(jnp.where(m2, 1, 0))

            def fire1(r, carry):
                idr = jnp.sum(jnp.where(i16 == r, v1, 0))
                pltpu.async_copy(tab_ref.at[r, pl.ds(idr, 1), :],
                                 rows_v.at[pl.ds(goff + r, 1), :], gsem)
                return carry
            lax.fori_loop(0, c1, fire1, 0)

            def fire2(r, carry):
                idr = jnp.sum(jnp.where(i16 == r, v2, 0))
                pltpu.async_copy(tab_ref.at[16 + r, pl.ds(idr, 1), :],
                                 rows_v.at[pl.ds(goff + c1 + r, 1), :], gsem)
                return carry
            lax.fori_loop(0, c2, fire2, 0)
            return f + 1, goff + c1 + c2

        def cond(st):
            f, goff = st
            return (f < limit) & (goff <= _G - _NT)
        return lax.while_loop(cond, body, (fstart, 0))

    def _drain(count, gsem):
        def body(r, carry):
            pltpu.make_async_copy(tab_ref.at[0, pl.ds(0, 1), :], dslot,
                                  gsem).wait()
            return carry
        lax.fori_loop(0, count, body, 0)

    f1_0, k0 = _fire(0, n0, rows0, sem_g0)
    f1_1, k1 = _fire(n0, ntot, rows1, sem_g1)

    def _p2(fstart, fend, roff0, rows_v, choff):
        def body(st):
            f, roff = st
            jl = _lane(fixpos, f)
            v1, m1, v2, m2 = _cand(jl)
            c = jnp.sum(jnp.where(m1, 1, 0)) + jnp.sum(jnp.where(m2, 1, 0))
            accs = tuple(suffix[pl.ds(c * _D + 16 * k, 16)]
                         for k in range(4))

            def inner(r, accs):
                return tuple(
                    a + rows_v[roff + r, pl.ds(16 * k, 16)]
                    for k, a in enumerate(accs))
            accs = lax.fori_loop(0, c, inner, accs)
            for k in range(4):
                out_v[jl - choff, pl.ds(16 * k, 16)] = accs[k] * _INV
            return f + 1, roff + c
        return lax.while_loop(lambda st: st[0] < fend, body,
                              (fstart, roff0))

    chunk_meta = [
        (0, f1_0, n0, k0, rows0, sem_g0, sem_o0),
        (n0, f1_1, ntot, k1, rows1, sem_g1, sem_o1),
    ]
    out_copies = []
    for ch in range(2):
        s_ch, f1_ch, e_ch, k_ch, rv, gsem, osem = chunk_meta[ch]
        if ch == 1:
            out_copies[0].wait()

        def fill_body(q, carry):
            for u in range(4):
                for k in range(4):
                    out_v[q * 4 + u, pl.ds(16 * k, 16)] = cst[k]
            return carry
        lax.fori_loop(0, _P // 4, fill_body, 0)

        _drain(k_ch, gsem)
        fdone, _ = _p2(s_ch, f1_ch, 0, rv, ch * _P)

        def rem_body(f):
            fn, kk = _fire(f, e_ch, rv, gsem)
            _drain(kk, gsem)
            fn2, _ = _p2(f, fn, 0, rv, ch * _P)
            return fn2
        lax.while_loop(lambda f: f < e_ch, rem_body, fdone)

        out_copies.append(
            pltpu.async_copy(out_v,
                             out_ref.at[bb, pl.ds(l0 + ch * _P, _P), :],
                             osem))
    out_copies[1].wait()


@functools.cache
def _sc_call():
    return pl.kernel(
        _sc_body,
        out_type=jax.ShapeDtypeStruct((_B, _L, _D), jnp.float32),
        mesh=plsc.VectorSubcoreMesh(core_axis_name="c", subcore_axis_name="s"),
        compiler_params=pltpu.CompilerParams(needs_layout_passes=False),
        scratch_types=[
            pltpu.VMEM((_NT, _W), jnp.int32),
            pltpu.VMEM((_P, _D), jnp.float32),
            pltpu.VMEM((_W + 16,), jnp.int32),
            pltpu.VMEM((_G, _D), jnp.float32),
            pltpu.VMEM((_G, _D), jnp.float32),
            pltpu.VMEM((_NT, _D), jnp.float32),
            pltpu.VMEM(((_NT + 1) * _D,), jnp.float32),
            pltpu.VMEM((1, _D), jnp.float32),
            pltpu.SemaphoreType.DMA,
            pltpu.SemaphoreType.DMA,
            pltpu.SemaphoreType.DMA,
            pltpu.SemaphoreType.DMA,
            pltpu.SemaphoreType.DMA,
            pltpu.SemaphoreType.DMA,
        ],
    )


def kernel(byte_ids, tables):
    byte_ids = byte_ids.astype(jnp.int32)
    ids = _hash_call(byte_ids)
    return _sc_call()(tables, ids).astype(tables.dtype)
